# baseline (device time: 811696 ns/iter reference)
import jax
import jax.numpy as jnp
from jax import lax
from jax.experimental import pallas as pl
from jax.experimental.pallas import tpu as pltpu

NBUF = 4
ROWS = 1024


def kernel(x):
    m, n2 = x.shape
    n = n2 // 2
    nchunk = m // ROWS

    def body(x_ref, out_ref, vmem, in_sems, out_sems, send_sem, recv_sem):
        my_x = lax.axis_index("x")
        my_y = lax.axis_index("y")
        my_z = lax.axis_index("z")
        partner = (1 - my_x, my_y, my_z)

        barrier = pltpu.get_barrier_semaphore()
        pl.semaphore_signal(
            barrier, inc=1, device_id=partner,
            device_id_type=pl.DeviceIdType.MESH,
        )
        pl.semaphore_wait(barrier, 1)

        rdma = pltpu.make_async_remote_copy(
            src_ref=x_ref.at[:, pl.ds((1 - my_x) * n, n)],
            dst_ref=out_ref.at[pl.ds(my_x * m, m), :],
            send_sem=send_sem,
            recv_sem=recv_sem,
            device_id=partner,
            device_id_type=pl.DeviceIdType.MESH,
        )
        rdma.start()

        out_cps = [None] * nchunk
        for c in range(nchunk):
            slot = c % NBUF
            if c >= NBUF:
                out_cps[c - NBUF].wait()
            in_cp = pltpu.make_async_copy(
                x_ref.at[pl.ds(c * ROWS, ROWS), pl.ds(my_x * n, n)],
                vmem.at[slot],
                in_sems.at[slot],
            )
            in_cp.start()
            in_cp.wait()
            out_cp = pltpu.make_async_copy(
                vmem.at[slot],
                out_ref.at[pl.ds(my_x * m + c * ROWS, ROWS), :],
                out_sems.at[slot],
            )
            out_cp.start()
            out_cps[c] = out_cp
        for c in range(nchunk - NBUF, nchunk):
            out_cps[c].wait()

        rdma.wait()

    return pl.pallas_call(
        body,
        out_shape=jax.ShapeDtypeStruct((2 * m, n), jnp.float32),
        in_specs=[pl.BlockSpec(memory_space=pl.ANY)],
        out_specs=pl.BlockSpec(memory_space=pl.ANY),
        scratch_shapes=[
            pltpu.VMEM((NBUF, ROWS, 1024), jnp.float32),
            pltpu.SemaphoreType.DMA((NBUF,)),
            pltpu.SemaphoreType.DMA((NBUF,)),
            pltpu.SemaphoreType.DMA,
            pltpu.SemaphoreType.DMA,
        ],
        compiler_params=pltpu.CompilerParams(collective_id=0),
    )(x)


# device time: 477917 ns/iter; 1.6984x vs baseline; 1.6984x over previous
import jax
import jax.numpy as jnp
from jax import lax
from jax.experimental import pallas as pl
from jax.experimental.pallas import tpu as pltpu

NBUF = 4
LROWS = 1024
CROWS = 512


def kernel(x):
    m, n2 = x.shape
    n = n2 // 2
    half = m // 2
    ncc = half // CROWS
    nlc = m // LROWS
    lpc = nlc // ncc

    def body(x_ref, out_ref, vmem, lin_sems, lout_sems,
             xs_sems, xr_sems, ys_sems, yr_sems):
        my_x = lax.axis_index("x")
        my_y = lax.axis_index("y")
        my_z = lax.axis_index("z")
        partner = (1 - my_x, my_y, my_z)
        ynbr = (my_x, 1 - my_y, my_z)

        barrier = pltpu.get_barrier_semaphore()
        for nbr in (partner, ynbr):
            pl.semaphore_signal(
                barrier, inc=1, device_id=nbr,
                device_id_type=pl.DeviceIdType.MESH,
            )
        pl.semaphore_wait(barrier, 2)

        sbase = my_y * half

        xr = []
        for c in range(ncc):
            rdma = pltpu.make_async_remote_copy(
                src_ref=x_ref.at[pl.ds(sbase + c * CROWS, CROWS),
                                 pl.ds((1 - my_x) * n, n)],
                dst_ref=out_ref.at[pl.ds(my_x * m + sbase + c * CROWS,
                                         CROWS), :],
                send_sem=xs_sems.at[c],
                recv_sem=xr_sems.at[c],
                device_id=partner,
                device_id_type=pl.DeviceIdType.MESH,
            )
            rdma.start()
            xr.append(rdma)

        lslot = 0
        lout_cps = [None] * nlc
        yfw = []
        for c in range(ncc):
            xr[c].wait_recv()
            row = (1 - my_x) * m + sbase + c * CROWS
            fwd = pltpu.make_async_remote_copy(
                src_ref=out_ref.at[pl.ds(row, CROWS), :],
                dst_ref=out_ref.at[pl.ds(row, CROWS), :],
                send_sem=ys_sems.at[c],
                recv_sem=yr_sems.at[c],
                device_id=ynbr,
                device_id_type=pl.DeviceIdType.MESH,
            )
            fwd.start()
            yfw.append(fwd)

            for _ in range(lpc):
                k = lslot
                slot = k % NBUF
                if k >= NBUF:
                    lout_cps[k - NBUF].wait()
                in_cp = pltpu.make_async_copy(
                    x_ref.at[pl.ds(k * LROWS, LROWS), pl.ds(my_x * n, n)],
                    vmem.at[slot],
                    lin_sems.at[slot],
                )
                in_cp.start()
                in_cp.wait()
                out_cp = pltpu.make_async_copy(
                    vmem.at[slot],
                    out_ref.at[pl.ds(my_x * m + k * LROWS, LROWS), :],
                    lout_sems.at[slot],
                )
                out_cp.start()
                lout_cps[k] = out_cp
                lslot += 1

        for c in range(ncc):
            xr[c].wait_send()
            yfw[c].wait()
        for k in range(nlc - NBUF, nlc):
            lout_cps[k].wait()

    return pl.pallas_call(
        body,
        out_shape=jax.ShapeDtypeStruct((2 * m, n), jnp.float32),
        in_specs=[pl.BlockSpec(memory_space=pl.ANY)],
        out_specs=pl.BlockSpec(memory_space=pl.ANY),
        scratch_shapes=[
            pltpu.VMEM((NBUF, LROWS, 1024), jnp.float32),
            pltpu.SemaphoreType.DMA((NBUF,)),
            pltpu.SemaphoreType.DMA((NBUF,)),
            pltpu.SemaphoreType.DMA((half // CROWS,)),
            pltpu.SemaphoreType.DMA((half // CROWS,)),
            pltpu.SemaphoreType.DMA((half // CROWS,)),
            pltpu.SemaphoreType.DMA((half // CROWS,)),
        ],
        compiler_params=pltpu.CompilerParams(collective_id=0),
    )(x)


# device time: 391103 ns/iter; 2.0754x vs baseline; 1.2220x over previous
import jax
import jax.numpy as jnp
from jax import lax
from jax.experimental import pallas as pl
from jax.experimental.pallas import tpu as pltpu

NBUF = 4
LROWS = 1024
CROWS = 512


def kernel(x):
    m, n2 = x.shape
    n = n2 // 2
    quarter = m // 4
    ncc = quarter // CROWS
    nlc = m // LROWS
    lpc = nlc // ncc

    def body(x_ref, out_ref, vmem, lin_sems, lout_sems,
             xs_sems, xr_sems, yfs_sems, yfr_sems, zfs_sems, zfr_sems,
             yrs_sems, yrr_sems, zrs_sems, zrr_sems):
        my_x = lax.axis_index("x")
        my_y = lax.axis_index("y")
        my_z = lax.axis_index("z")
        partner = (1 - my_x, my_y, my_z)
        ynbr = (my_x, 1 - my_y, my_z)
        znbr = (my_x, my_y, 1 - my_z)

        barrier = pltpu.get_barrier_semaphore()
        for nbr in (partner, ynbr, znbr):
            pl.semaphore_signal(
                barrier, inc=1, device_id=nbr,
                device_id_type=pl.DeviceIdType.MESH,
            )
        pl.semaphore_wait(barrier, 3)

        def qb(y, z):
            return (2 * y + z) * quarter

        rbase = (1 - my_x) * m
        qmine = qb(my_y, my_z)

        xr = []
        for c in range(ncc):
            rdma = pltpu.make_async_remote_copy(
                src_ref=x_ref.at[pl.ds(qmine + c * CROWS, CROWS),
                                 pl.ds((1 - my_x) * n, n)],
                dst_ref=out_ref.at[pl.ds(my_x * m + qmine + c * CROWS,
                                         CROWS), :],
                send_sem=xs_sems.at[c],
                recv_sem=xr_sems.at[c],
                device_id=partner,
                device_id_type=pl.DeviceIdType.MESH,
            )
            rdma.start()
            xr.append(rdma)

        def fwd(row, dev, send_sems, recv_sems, idx):
            r = pltpu.make_async_remote_copy(
                src_ref=out_ref.at[pl.ds(row, CROWS), :],
                dst_ref=out_ref.at[pl.ds(row, CROWS), :],
                send_sem=send_sems.at[idx],
                recv_sem=recv_sems.at[idx],
                device_id=dev,
                device_id_type=pl.DeviceIdType.MESH,
            )
            r.start()
            return r

        yfw = [None] * ncc
        zfw = [None] * ncc
        yrl = [None] * (ncc // 2)
        zrl = [None] * (ncc // 2)
        lslot = 0
        lout_cps = [None] * nlc

        def process_nbr(j):
            zfw[j].wait_recv()
            if j % 2 == 0:
                row = rbase + qb(my_y, 1 - my_z) + j * CROWS
                yrl[j // 2] = fwd(row, ynbr, yrs_sems, yrr_sems, j // 2)
            yfw[j].wait_recv()
            if j % 2 == 1:
                row = rbase + qb(1 - my_y, my_z) + j * CROWS
                zrl[j // 2] = fwd(row, znbr, zrs_sems, zrr_sems, j // 2)

        for c in range(ncc):
            xr[c].wait_recv()
            row = rbase + qmine + c * CROWS
            yfw[c] = fwd(row, ynbr, yfs_sems, yfr_sems, c)
            zfw[c] = fwd(row, znbr, zfs_sems, zfr_sems, c)

            for _ in range(lpc):
                k = lslot
                slot = k % NBUF
                if k >= NBUF:
                    lout_cps[k - NBUF].wait()
                in_cp = pltpu.make_async_copy(
                    x_ref.at[pl.ds(k * LROWS, LROWS), pl.ds(my_x * n, n)],
                    vmem.at[slot],
                    lin_sems.at[slot],
                )
                in_cp.start()
                in_cp.wait()
                out_cp = pltpu.make_async_copy(
                    vmem.at[slot],
                    out_ref.at[pl.ds(my_x * m + k * LROWS, LROWS), :],
                    lout_sems.at[slot],
                )
                out_cp.start()
                lout_cps[k] = out_cp
                lslot += 1

            if c >= 1:
                process_nbr(c - 1)
        process_nbr(ncc - 1)

        for c in range(ncc):
            xr[c].wait_send()
            yfw[c].wait_send()
            zfw[c].wait_send()
        for i in range(ncc // 2):
            yrl[i].wait()
            zrl[i].wait()
        for k in range(nlc - NBUF, nlc):
            lout_cps[k].wait()

    return pl.pallas_call(
        body,
        out_shape=jax.ShapeDtypeStruct((2 * m, n), jnp.float32),
        in_specs=[pl.BlockSpec(memory_space=pl.ANY)],
        out_specs=pl.BlockSpec(memory_space=pl.ANY),
        scratch_shapes=[
            pltpu.VMEM((NBUF, LROWS, 1024), jnp.float32),
            pltpu.SemaphoreType.DMA((NBUF,)),
            pltpu.SemaphoreType.DMA((NBUF,)),
            pltpu.SemaphoreType.DMA((ncc,)),
            pltpu.SemaphoreType.DMA((ncc,)),
            pltpu.SemaphoreType.DMA((ncc,)),
            pltpu.SemaphoreType.DMA((ncc,)),
            pltpu.SemaphoreType.DMA((ncc,)),
            pltpu.SemaphoreType.DMA((ncc,)),
            pltpu.SemaphoreType.DMA((ncc // 2,)),
            pltpu.SemaphoreType.DMA((ncc // 2,)),
            pltpu.SemaphoreType.DMA((ncc // 2,)),
            pltpu.SemaphoreType.DMA((ncc // 2,)),
        ],
        compiler_params=pltpu.CompilerParams(collective_id=0),
    )(x)


# device time: 368536 ns/iter; 2.2025x vs baseline; 1.0612x over previous
import jax
import jax.numpy as jnp
from jax import lax
from jax.experimental import pallas as pl
from jax.experimental.pallas import tpu as pltpu

NBUF = 4
LROWS = 1024
CROWS = 512
NDIR = 2


def kernel(x):
    m, n2 = x.shape
    n = n2 // 2
    quarter = m // 4
    ncc = quarter // CROWS
    nlc = m // LROWS
    lpc = nlc // ncc

    def body(x_ref, out_ref, vmem, lin_sems, lout_sems,
             xs_sems, xr_sems, xds_sems, xdr_sems,
             yfs_sems, yfr_sems, zfs_sems, zfr_sems,
             yrs_sems, yrr_sems, zrs_sems, zrr_sems):
        my_x = lax.axis_index("x")
        my_y = lax.axis_index("y")
        my_z = lax.axis_index("z")
        partner = (1 - my_x, my_y, my_z)
        ynbr = (my_x, 1 - my_y, my_z)
        znbr = (my_x, my_y, 1 - my_z)

        barrier = pltpu.get_barrier_semaphore()
        for nbr in (partner, ynbr, znbr):
            pl.semaphore_signal(
                barrier, inc=1, device_id=nbr,
                device_id_type=pl.DeviceIdType.MESH,
            )
        pl.semaphore_wait(barrier, 3)

        def qb(y, z):
            return (2 * y + z) * quarter

        rbase = (1 - my_x) * m
        qmine = qb(my_y, my_z)
        qdiag = qb(1 - my_y, 1 - my_z)

        xr = []
        for c in range(ncc):
            rdma = pltpu.make_async_remote_copy(
                src_ref=x_ref.at[pl.ds(qmine + c * CROWS, CROWS),
                                 pl.ds((1 - my_x) * n, n)],
                dst_ref=out_ref.at[pl.ds(my_x * m + qmine + c * CROWS,
                                         CROWS), :],
                send_sem=xs_sems.at[c],
                recv_sem=xr_sems.at[c],
                device_id=partner,
                device_id_type=pl.DeviceIdType.MESH,
            )
            rdma.start()
            xr.append(rdma)
        xdiag = []
        for c in range(NDIR):
            rdma = pltpu.make_async_remote_copy(
                src_ref=x_ref.at[pl.ds(qdiag + c * CROWS, CROWS),
                                 pl.ds((1 - my_x) * n, n)],
                dst_ref=out_ref.at[pl.ds(my_x * m + qdiag + c * CROWS,
                                         CROWS), :],
                send_sem=xds_sems.at[c],
                recv_sem=xdr_sems.at[c],
                device_id=partner,
                device_id_type=pl.DeviceIdType.MESH,
            )
            rdma.start()
            xdiag.append(rdma)

        def fwd(row, dev, send_sems, recv_sems, idx):
            r = pltpu.make_async_remote_copy(
                src_ref=out_ref.at[pl.ds(row, CROWS), :],
                dst_ref=out_ref.at[pl.ds(row, CROWS), :],
                send_sem=send_sems.at[idx],
                recv_sem=recv_sems.at[idx],
                device_id=dev,
                device_id_type=pl.DeviceIdType.MESH,
            )
            r.start()
            return r

        yfw = [None] * ncc
        zfw = [None] * ncc
        nrel = (ncc - NDIR) // 2
        yrl = [None] * nrel
        zrl = [None] * nrel
        lslot = 0
        lout_cps = [None] * nlc

        def process_nbr(j):
            zfw[j].wait_recv()
            if j % 2 == 0 and j >= NDIR:
                row = rbase + qb(my_y, 1 - my_z) + j * CROWS
                yrl[(j - NDIR) // 2] = fwd(
                    row, ynbr, yrs_sems, yrr_sems, (j - NDIR) // 2)
            yfw[j].wait_recv()
            if j % 2 == 1 and j >= NDIR:
                row = rbase + qb(1 - my_y, my_z) + j * CROWS
                zrl[(j - NDIR - 1) // 2] = fwd(
                    row, znbr, zrs_sems, zrr_sems, (j - NDIR - 1) // 2)

        for c in range(ncc):
            xr[c].wait_recv()
            row = rbase + qmine + c * CROWS
            yfw[c] = fwd(row, ynbr, yfs_sems, yfr_sems, c)
            zfw[c] = fwd(row, znbr, zfs_sems, zfr_sems, c)

            if c >= 1:
                process_nbr(c - 1)

            for _ in range(lpc):
                k = lslot
                slot = k % NBUF
                if k >= NBUF:
                    lout_cps[k - NBUF].wait()
                in_cp = pltpu.make_async_copy(
                    x_ref.at[pl.ds(k * LROWS, LROWS), pl.ds(my_x * n, n)],
                    vmem.at[slot],
                    lin_sems.at[slot],
                )
                in_cp.start()
                in_cp.wait()
                out_cp = pltpu.make_async_copy(
                    vmem.at[slot],
                    out_ref.at[pl.ds(my_x * m + k * LROWS, LROWS), :],
                    lout_sems.at[slot],
                )
                out_cp.start()
                lout_cps[k] = out_cp
                lslot += 1
        process_nbr(ncc - 1)

        for c in range(ncc):
            xr[c].wait_send()
            yfw[c].wait_send()
            zfw[c].wait_send()
        for c in range(NDIR):
            xdiag[c].wait()
        for i in range(nrel):
            yrl[i].wait()
            zrl[i].wait()
        for k in range(nlc - NBUF, nlc):
            lout_cps[k].wait()

    return pl.pallas_call(
        body,
        out_shape=jax.ShapeDtypeStruct((2 * m, n), jnp.float32),
        in_specs=[pl.BlockSpec(memory_space=pl.ANY)],
        out_specs=pl.BlockSpec(memory_space=pl.ANY),
        scratch_shapes=[
            pltpu.VMEM((NBUF, LROWS, 1024), jnp.float32),
            pltpu.SemaphoreType.DMA((NBUF,)),
            pltpu.SemaphoreType.DMA((NBUF,)),
            pltpu.SemaphoreType.DMA((ncc,)),
            pltpu.SemaphoreType.DMA((ncc,)),
            pltpu.SemaphoreType.DMA((max(NDIR, 1),)),
            pltpu.SemaphoreType.DMA((max(NDIR, 1),)),
            pltpu.SemaphoreType.DMA((ncc,)),
            pltpu.SemaphoreType.DMA((ncc,)),
            pltpu.SemaphoreType.DMA((ncc,)),
            pltpu.SemaphoreType.DMA((ncc,)),
            pltpu.SemaphoreType.DMA(((ncc - NDIR) // 2,)),
            pltpu.SemaphoreType.DMA(((ncc - NDIR) // 2,)),
            pltpu.SemaphoreType.DMA(((ncc - NDIR) // 2,)),
            pltpu.SemaphoreType.DMA(((ncc - NDIR) // 2,)),
        ],
        compiler_params=pltpu.CompilerParams(collective_id=0),
    )(x)


# device time: 357562 ns/iter; 2.2701x vs baseline; 1.0307x over previous
import jax
import jax.numpy as jnp
from jax import lax
from jax.experimental import pallas as pl
from jax.experimental.pallas import tpu as pltpu

NBUF = 4
LROWS = 1024
CROWS = 256
NDIR = 5


def kernel(x):
    m, n2 = x.shape
    n = n2 // 2
    quarter = m // 4
    ncc = quarter // CROWS
    nlc = m // LROWS
    lpc = nlc // ncc

    ylist = [j for j in range(ncc) if j >= NDIR and j % 2 == 0]
    zlist = [j for j in range(ncc) if j >= NDIR and j % 2 == 1]
    yidx = {j: i for i, j in enumerate(ylist)}
    zidx = {j: i for i, j in enumerate(zlist)}

    def body(x_ref, out_ref, vmem, lin_sems, lout_sems,
             xs_sems, xr_sems, xds_sems, xdr_sems,
             yfs_sems, yfr_sems, zfs_sems, zfr_sems,
             yrs_sems, yrr_sems, zrs_sems, zrr_sems):
        my_x = lax.axis_index("x")
        my_y = lax.axis_index("y")
        my_z = lax.axis_index("z")
        partner = (1 - my_x, my_y, my_z)
        ynbr = (my_x, 1 - my_y, my_z)
        znbr = (my_x, my_y, 1 - my_z)

        barrier = pltpu.get_barrier_semaphore()
        for nbr in (partner, ynbr, znbr):
            pl.semaphore_signal(
                barrier, inc=1, device_id=nbr,
                device_id_type=pl.DeviceIdType.MESH,
            )
        pl.semaphore_wait(barrier, 3)

        def qb(y, z):
            return (2 * y + z) * quarter

        rbase = (1 - my_x) * m
        qmine = qb(my_y, my_z)
        qdiag = qb(1 - my_y, 1 - my_z)

        xr = []
        for c in range(ncc):
            rdma = pltpu.make_async_remote_copy(
                src_ref=x_ref.at[pl.ds(qmine + c * CROWS, CROWS),
                                 pl.ds((1 - my_x) * n, n)],
                dst_ref=out_ref.at[pl.ds(my_x * m + qmine + c * CROWS,
                                         CROWS), :],
                send_sem=xs_sems.at[c],
                recv_sem=xr_sems.at[c],
                device_id=partner,
                device_id_type=pl.DeviceIdType.MESH,
            )
            rdma.start()
            xr.append(rdma)
        xdiag = []
        for c in range(NDIR):
            rdma = pltpu.make_async_remote_copy(
                src_ref=x_ref.at[pl.ds(qdiag + c * CROWS, CROWS),
                                 pl.ds((1 - my_x) * n, n)],
                dst_ref=out_ref.at[pl.ds(my_x * m + qdiag + c * CROWS,
                                         CROWS), :],
                send_sem=xds_sems.at[c],
                recv_sem=xdr_sems.at[c],
                device_id=partner,
                device_id_type=pl.DeviceIdType.MESH,
            )
            rdma.start()
            xdiag.append(rdma)

        def fwd(row, dev, send_sems, recv_sems, idx):
            r = pltpu.make_async_remote_copy(
                src_ref=out_ref.at[pl.ds(row, CROWS), :],
                dst_ref=out_ref.at[pl.ds(row, CROWS), :],
                send_sem=send_sems.at[idx],
                recv_sem=recv_sems.at[idx],
                device_id=dev,
                device_id_type=pl.DeviceIdType.MESH,
            )
            r.start()
            return r

        yfw = [None] * ncc
        zfw = [None] * ncc
        yrl = [None] * len(ylist)
        zrl = [None] * len(zlist)
        lslot = 0
        lout_cps = [None] * nlc

        def process_nbr(j):
            zfw[j].wait_recv()
            if j in yidx:
                row = rbase + qb(my_y, 1 - my_z) + j * CROWS
                yrl[yidx[j]] = fwd(row, ynbr, yrs_sems, yrr_sems, yidx[j])
            yfw[j].wait_recv()
            if j in zidx:
                row = rbase + qb(1 - my_y, my_z) + j * CROWS
                zrl[zidx[j]] = fwd(row, znbr, zrs_sems, zrr_sems, zidx[j])

        for c in range(ncc):
            xr[c].wait_recv()
            row = rbase + qmine + c * CROWS
            yfw[c] = fwd(row, ynbr, yfs_sems, yfr_sems, c)
            zfw[c] = fwd(row, znbr, zfs_sems, zfr_sems, c)

            if c >= 1:
                process_nbr(c - 1)

            for _ in range(lpc):
                k = lslot
                slot = k % NBUF
                if k >= NBUF:
                    lout_cps[k - NBUF].wait()
                in_cp = pltpu.make_async_copy(
                    x_ref.at[pl.ds(k * LROWS, LROWS), pl.ds(my_x * n, n)],
                    vmem.at[slot],
                    lin_sems.at[slot],
                )
                in_cp.start()
                in_cp.wait()
                out_cp = pltpu.make_async_copy(
                    vmem.at[slot],
                    out_ref.at[pl.ds(my_x * m + k * LROWS, LROWS), :],
                    lout_sems.at[slot],
                )
                out_cp.start()
                lout_cps[k] = out_cp
                lslot += 1
        process_nbr(ncc - 1)

        for c in range(ncc):
            xr[c].wait_send()
            yfw[c].wait_send()
            zfw[c].wait_send()
        for c in range(NDIR):
            xdiag[c].wait()
        for i in range(len(ylist)):
            yrl[i].wait()
        for i in range(len(zlist)):
            zrl[i].wait()
        for k in range(nlc - NBUF, nlc):
            lout_cps[k].wait()

    return pl.pallas_call(
        body,
        out_shape=jax.ShapeDtypeStruct((2 * m, n), jnp.float32),
        in_specs=[pl.BlockSpec(memory_space=pl.ANY)],
        out_specs=pl.BlockSpec(memory_space=pl.ANY),
        scratch_shapes=[
            pltpu.VMEM((NBUF, LROWS, 1024), jnp.float32),
            pltpu.SemaphoreType.DMA((NBUF,)),
            pltpu.SemaphoreType.DMA((NBUF,)),
            pltpu.SemaphoreType.DMA((ncc,)),
            pltpu.SemaphoreType.DMA((ncc,)),
            pltpu.SemaphoreType.DMA((NDIR,)),
            pltpu.SemaphoreType.DMA((NDIR,)),
            pltpu.SemaphoreType.DMA((ncc,)),
            pltpu.SemaphoreType.DMA((ncc,)),
            pltpu.SemaphoreType.DMA((ncc,)),
            pltpu.SemaphoreType.DMA((ncc,)),
            pltpu.SemaphoreType.DMA((len(ylist),)),
            pltpu.SemaphoreType.DMA((len(ylist),)),
            pltpu.SemaphoreType.DMA((len(zlist),)),
            pltpu.SemaphoreType.DMA((len(zlist),)),
        ],
        compiler_params=pltpu.CompilerParams(collective_id=0),
    )(x)


# device time: 350164 ns/iter; 2.3180x vs baseline; 1.0211x over previous
import jax
import jax.numpy as jnp
from jax import lax
from jax.experimental import pallas as pl
from jax.experimental.pallas import tpu as pltpu

NBUF = 4
LROWS = 1024
CROWS = 128
NDIR = 11


def kernel(x):
    m, n2 = x.shape
    n = n2 // 2
    quarter = m // 4
    ncc = quarter // CROWS
    nlc = m // LROWS

    ylist = [j for j in range(ncc) if j >= NDIR and j % 2 == 0]
    zlist = [j for j in range(ncc) if j >= NDIR and j % 2 == 1]
    yidx = {j: i for i, j in enumerate(ylist)}
    zidx = {j: i for i, j in enumerate(zlist)}

    def body(x_ref, out_ref, vmem, lin_sems, lout_sems,
             xs_sems, xr_sems, xds_sems, xdr_sems,
             yfs_sems, yfr_sems, zfs_sems, zfr_sems,
             yrs_sems, yrr_sems, zrs_sems, zrr_sems):
        my_x = lax.axis_index("x")
        my_y = lax.axis_index("y")
        my_z = lax.axis_index("z")
        partner = (1 - my_x, my_y, my_z)
        ynbr = (my_x, 1 - my_y, my_z)
        znbr = (my_x, my_y, 1 - my_z)

        barrier = pltpu.get_barrier_semaphore()
        for nbr in (partner, ynbr, znbr):
            pl.semaphore_signal(
                barrier, inc=1, device_id=nbr,
                device_id_type=pl.DeviceIdType.MESH,
            )
        pl.semaphore_wait(barrier, 3)

        def qb(y, z):
            return (2 * y + z) * quarter

        rbase = (1 - my_x) * m
        qmine = qb(my_y, my_z)
        qdiag = qb(1 - my_y, 1 - my_z)

        xr = []
        for c in range(ncc):
            rdma = pltpu.make_async_remote_copy(
                src_ref=x_ref.at[pl.ds(qmine + c * CROWS, CROWS),
                                 pl.ds((1 - my_x) * n, n)],
                dst_ref=out_ref.at[pl.ds(my_x * m + qmine + c * CROWS,
                                         CROWS), :],
                send_sem=xs_sems.at[c],
                recv_sem=xr_sems.at[c],
                device_id=partner,
                device_id_type=pl.DeviceIdType.MESH,
            )
            rdma.start()
            xr.append(rdma)
        xdiag = []
        for c in range(NDIR):
            rdma = pltpu.make_async_remote_copy(
                src_ref=x_ref.at[pl.ds(qdiag + c * CROWS, CROWS),
                                 pl.ds((1 - my_x) * n, n)],
                dst_ref=out_ref.at[pl.ds(my_x * m + qdiag + c * CROWS,
                                         CROWS), :],
                send_sem=xds_sems.at[c],
                recv_sem=xdr_sems.at[c],
                device_id=partner,
                device_id_type=pl.DeviceIdType.MESH,
            )
            rdma.start()
            xdiag.append(rdma)

        def fwd(row, dev, send_sems, recv_sems, idx):
            r = pltpu.make_async_remote_copy(
                src_ref=out_ref.at[pl.ds(row, CROWS), :],
                dst_ref=out_ref.at[pl.ds(row, CROWS), :],
                send_sem=send_sems.at[idx],
                recv_sem=recv_sems.at[idx],
                device_id=dev,
                device_id_type=pl.DeviceIdType.MESH,
            )
            r.start()
            return r

        yfw = [None] * ncc
        zfw = [None] * ncc
        yrl = [None] * len(ylist)
        zrl = [None] * len(zlist)
        lslot = 0
        lout_cps = [None] * nlc

        def process_nbr(j):
            zfw[j].wait_recv()
            if j in yidx:
                row = rbase + qb(my_y, 1 - my_z) + j * CROWS
                yrl[yidx[j]] = fwd(row, ynbr, yrs_sems, yrr_sems, yidx[j])
            yfw[j].wait_recv()
            if j in zidx:
                row = rbase + qb(1 - my_y, my_z) + j * CROWS
                zrl[zidx[j]] = fwd(row, znbr, zrs_sems, zrr_sems, zidx[j])

        for c in range(ncc):
            xr[c].wait_recv()
            row = rbase + qmine + c * CROWS
            yfw[c] = fwd(row, ynbr, yfs_sems, yfr_sems, c)
            zfw[c] = fwd(row, znbr, zfs_sems, zfr_sems, c)

            if c >= 1:
                process_nbr(c - 1)

            for k in ([lslot] if c % 2 == 0 and lslot < nlc else []):
                slot = k % NBUF
                if k >= NBUF:
                    lout_cps[k - NBUF].wait()
                in_cp = pltpu.make_async_copy(
                    x_ref.at[pl.ds(k * LROWS, LROWS), pl.ds(my_x * n, n)],
                    vmem.at[slot],
                    lin_sems.at[slot],
                )
                in_cp.start()
                in_cp.wait()
                out_cp = pltpu.make_async_copy(
                    vmem.at[slot],
                    out_ref.at[pl.ds(my_x * m + k * LROWS, LROWS), :],
                    lout_sems.at[slot],
                )
                out_cp.start()
                lout_cps[k] = out_cp
                lslot += 1
        process_nbr(ncc - 1)

        for c in range(ncc):
            xr[c].wait_send()
            yfw[c].wait_send()
            zfw[c].wait_send()
        for c in range(NDIR):
            xdiag[c].wait()
        for i in range(len(ylist)):
            yrl[i].wait()
        for i in range(len(zlist)):
            zrl[i].wait()
        for k in range(nlc - NBUF, nlc):
            lout_cps[k].wait()

    return pl.pallas_call(
        body,
        out_shape=jax.ShapeDtypeStruct((2 * m, n), jnp.float32),
        in_specs=[pl.BlockSpec(memory_space=pl.ANY)],
        out_specs=pl.BlockSpec(memory_space=pl.ANY),
        scratch_shapes=[
            pltpu.VMEM((NBUF, LROWS, 1024), jnp.float32),
            pltpu.SemaphoreType.DMA((NBUF,)),
            pltpu.SemaphoreType.DMA((NBUF,)),
            pltpu.SemaphoreType.DMA((ncc,)),
            pltpu.SemaphoreType.DMA((ncc,)),
            pltpu.SemaphoreType.DMA((NDIR,)),
            pltpu.SemaphoreType.DMA((NDIR,)),
            pltpu.SemaphoreType.DMA((ncc,)),
            pltpu.SemaphoreType.DMA((ncc,)),
            pltpu.SemaphoreType.DMA((ncc,)),
            pltpu.SemaphoreType.DMA((ncc,)),
            pltpu.SemaphoreType.DMA((len(ylist),)),
            pltpu.SemaphoreType.DMA((len(ylist),)),
            pltpu.SemaphoreType.DMA((len(zlist),)),
            pltpu.SemaphoreType.DMA((len(zlist),)),
        ],
        compiler_params=pltpu.CompilerParams(collective_id=0),
    )(x)
